# baseline (speedup 1.0000x reference)
import functools

import jax
import jax.numpy as jnp
from jax import lax
from jax.experimental import pallas as pl
from jax.experimental.pallas import tpu as pltpu
from jax.experimental.pallas import tpu_sc as plsc

VOCAB = 1000000
D = 64
N = 4096 * 200

NC = 2
NS = 16
NW = NC * NS
ROWS_PER_W = N // NW
SUB = 128
CHUNK = 512
N_SUB = CHUNK // SUB
N_CHUNKS = ROWS_PER_W // CHUNK
RT = N // SUB
CT = D // 8

_LN2 = 0.6931471805599453

_sc_mesh = plsc.VectorSubcoreMesh(core_axis_name="c", subcore_axis_name="s")


def _ln(v):
    bits = plsc.bitcast(v, jnp.int32)
    e = ((bits >> 23) & 0xFF) - 127
    m = plsc.bitcast((bits & 0x007FFFFF) | 0x3F800000, jnp.float32)
    z = (m - 1.0) / (m + 1.0)
    z2 = z * z
    p = 1.0 / 7.0 + z2 * (1.0 / 9.0)
    p = 1.0 / 5.0 + z2 * p
    p = 1.0 / 3.0 + z2 * p
    lnm = 2.0 * z * (1.0 + z2 * p)
    return lnm + e.astype(jnp.float32) * _LN2


@functools.partial(
    pl.kernel,
    mesh=_sc_mesh,
    out_type=(
        jax.ShapeDtypeStruct((CT, RT, 8 * SUB), jnp.float32),
        jax.ShapeDtypeStruct((NW, 16), jnp.float32),
    ),
    scratch_types=[
        pltpu.VMEM((N_SUB, SUB), jnp.int32),
        pltpu.VMEM((CHUNK,), jnp.int32),
        pltpu.VMEM((CHUNK, D), jnp.float32),
        pltpu.VMEM((16,), jnp.float32),
        [pltpu.VMEM((D * SUB,), jnp.float32) for _ in range(N_SUB)],
        pltpu.SemaphoreType.DMA,
        pltpu.SemaphoreType.DMA,
    ],
    compiler_params=pltpu.CompilerParams(
        use_tc_tiling_on_sc=False, needs_layout_passes=False),
)
def _sc_embed_ce(idx_hbm, tgt_hbm, table_hbm, out_hbm, part_hbm,
                 idx_v, tgt_v, buf, accv, tbufs, sem, sem_t):
    wid = lax.axis_index("s") * NC + lax.axis_index("c")
    grp0 = wid * (ROWS_PER_W // SUB)
    accv[...] = jnp.zeros((16,), jnp.float32)
    lane = lax.iota(jnp.int32, 16)
    colbase = [lane * SUB + (16 * k * SUB) for k in range(4)]

    def chunk_body(c, carry):
        g = grp0 + c * N_SUB
        pltpu.sync_copy(idx_hbm.at[pl.ds(g, N_SUB)], idx_v)
        pltpu.sync_copy(tgt_hbm.at[pl.ds(g * SUB, CHUNK)], tgt_v)
        gh = [
            pltpu.async_copy(
                table_hbm.at[idx_v.at[j]],
                buf.at[pl.ds(j * SUB, SUB)],
                sem,
            )
            for j in range(N_SUB)
        ]
        for h in gh:
            h.wait()

        handles = []
        for b in range(N_SUB):
            tb = tbufs[b]

            def grp_body(gi, carry2, _b=b, _tb=tb):
                row0 = _b * SUB + gi * 16
                tgt16 = tgt_v[pl.ds(row0, 16)]
                s_vec = jnp.zeros((16,), jnp.float32)
                for r in range(16):
                    v0 = buf[row0 + r, pl.ds(0, 16)]
                    v1 = buf[row0 + r, pl.ds(16, 16)]
                    v2 = buf[row0 + r, pl.ds(32, 16)]
                    v3 = buf[row0 + r, pl.ds(48, 16)]
                    q = gi * 16 + r
                    plsc.store_scatter(_tb, [colbase[0] + q], v0)
                    plsc.store_scatter(_tb, [colbase[1] + q], v1)
                    plsc.store_scatter(_tb, [colbase[2] + q], v2)
                    plsc.store_scatter(_tb, [colbase[3] + q], v3)
                    e = (jnp.exp(v0) + jnp.exp(v1)) + (jnp.exp(v2) + jnp.exp(v3))
                    s_vec = jnp.where(lane == r, jnp.sum(e), s_vec)
                base = row0 + lane
                picked = plsc.load_gather(buf, [base, tgt16])
                accv[...] = accv[...] + (_ln(s_vec) - picked)
                return carry2

            lax.fori_loop(0, SUB // 16, grp_body, 0)
            rt = grp0 + c * N_SUB + b
            for ct in range(CT):
                handles.append(
                    pltpu.async_copy(
                        tb.at[pl.ds(ct * 8 * SUB, 8 * SUB)],
                        out_hbm.at[ct, rt],
                        sem_t,
                    )
                )
        for h in handles:
            h.wait()
        return carry

    lax.fori_loop(0, N_CHUNKS, chunk_body, 0)
    pltpu.sync_copy(accv, part_hbm.at[wid])


def kernel(inputs, targets, wte):
    idx2 = (inputs.astype(jnp.int32).reshape(-1) * 2).reshape(N // SUB, SUB)
    tgt = targets.astype(jnp.int32).reshape(N)
    table = jnp.pad(wte, ((0, 0), (0, 128 - D))).reshape(2 * VOCAB, D)
    out3d, partials = _sc_embed_ce(idx2, tgt, table)
    logits2 = (
        out3d.reshape(CT, RT, 8, SUB)
        .transpose(1, 3, 0, 2)
        .reshape(N, D)
    )
    loss = jnp.sum(partials) * (1.0 / N)
    return (logits2, loss)
